# Initial kernel scaffold; baseline (speedup 1.0000x reference)
import jax
import jax.numpy as jnp
from jax.experimental import pallas as pl


def kernel(x, edge_index, batch, W1, b1, W2, b2, W3, b3, W4, b4, W5, b5, Wm1, bm1, Wm2, bm2):
    raise NotImplementedError("write your pallas kernel here")


# speedup vs baseline: 6.2539x; 6.2539x over previous
import functools

import jax
import jax.numpy as jnp
from jax import lax
from jax.experimental import pallas as pl
from jax.experimental.pallas import tpu as pltpu
from jax.experimental.pallas import tpu_sc as plsc

N = 10000
NP = 10240
E = 320000
CHUNK = 128
NCHUNKS = 2560
EPAD = NCHUNKS * CHUNK
NSUB = 16
NCORE = 2
CPW = NCHUNKS // (NSUB * NCORE)
SB = 16
H = 128
NHID = 256
B = 256
RPS = NP // NSUB
DUMMY = N + 16
BLK = 1024
NBLK = NP // BLK

f32 = jnp.float32
i32 = jnp.int32
_HIGH = lax.Precision.HIGHEST


def _fill(ref, nrows, width, value):
    def body(i, carry):
        for j in range(width // 16):
            ref[i, pl.ds(j * 16, 16)] = jnp.full((16,), value, f32)
        return carry
    lax.fori_loop(0, nrows, body, 0)


def _sc_deg(col2d):
    mesh = plsc.VectorSubcoreMesh(core_axis_name="c", subcore_axis_name="s")

    @functools.partial(
        pl.kernel,
        out_type=jax.ShapeDtypeStruct((NCORE, NP, H), f32),
        mesh=mesh,
        scratch_types=[
            pltpu.VMEM((CPW, CHUNK), i32),
            pltpu.VMEM((CHUNK, H), f32),
            pltpu.VMEM((CHUNK, H), f32),
            pltpu.VMEM_SHARED((NP, H), f32),
        ],
    )
    def k(col_hbm, out_hbm, col_v, zb, ob, acc):
        c = lax.axis_index("c")
        s = lax.axis_index("s")
        wid = c * NSUB + s
        pltpu.sync_copy(col_hbm.at[pl.ds(wid * CPW, CPW)], col_v)
        _fill(zb, CHUNK, H, 0.0)
        _fill(ob, CHUNK, H, 1.0)
        for j in range(RPS // CHUNK):
            pltpu.sync_copy(zb, acc.at[pl.ds(s * RPS + j * CHUNK, CHUNK)])
        plsc.subcore_barrier()

        def body(i, carry):
            pltpu.sync_copy(ob, acc.at[col_v.at[i]], add=True)
            return carry
        lax.fori_loop(0, CPW, body, 0)
        plsc.subcore_barrier()
        for j in range(RPS // CHUNK):
            off = s * RPS + j * CHUNK
            pltpu.sync_copy(acc.at[pl.ds(off, CHUNK)], zb)
            pltpu.sync_copy(zb, out_hbm.at[c, pl.ds(off, CHUNK)])

    return k(col2d)


def _sc_agg(hs, row2d, col2d):
    mesh = plsc.VectorSubcoreMesh(core_axis_name="c", subcore_axis_name="s")

    @functools.partial(
        pl.kernel,
        out_type=jax.ShapeDtypeStruct((NCORE, NP, H), f32),
        mesh=mesh,
        scratch_types=[
            pltpu.VMEM((SB, CHUNK), i32),
            pltpu.VMEM((SB, CHUNK), i32),
            pltpu.VMEM((CHUNK, H), f32),
            pltpu.VMEM((CHUNK, H), f32),
            pltpu.VMEM_SHARED((NP, H), f32),
            pltpu.SemaphoreType.DMA,
            pltpu.SemaphoreType.DMA,
        ],
    )
    def k(hs_hbm, row_hbm, col_hbm, out_hbm, row_v, col_v, buf_a, buf_b,
          acc, sem_a, sem_b):
        c = lax.axis_index("c")
        s = lax.axis_index("s")
        wid = c * NSUB + s
        _fill(buf_a, CHUNK, H, 0.0)
        for j in range(RPS // CHUNK):
            pltpu.sync_copy(buf_a, acc.at[pl.ds(s * RPS + j * CHUNK, CHUNK)])
        plsc.subcore_barrier()

        for t in range(CPW // SB):
            base = wid * CPW + t * SB
            pltpu.sync_copy(row_hbm.at[pl.ds(base, SB)], row_v)
            pltpu.sync_copy(col_hbm.at[pl.ds(base, SB)], col_v)
            pltpu.async_copy(hs_hbm.at[row_v.at[0]], buf_a, sem_a).wait()

            def body(i, carry):
                cp = pltpu.async_copy(hs_hbm.at[row_v.at[i + 1]], buf_b, sem_b)
                pltpu.sync_copy(buf_a, acc.at[col_v.at[i]], add=True)
                cp.wait()
                cp = pltpu.async_copy(hs_hbm.at[row_v.at[i + 2]], buf_a, sem_a)
                pltpu.sync_copy(buf_b, acc.at[col_v.at[i + 1]], add=True)
                cp.wait()
                return carry
            lax.fori_loop(0, SB // 2 - 1, lambda u, cr: body(2 * u, cr), 0)
            cp = pltpu.async_copy(hs_hbm.at[row_v.at[SB - 1]], buf_b, sem_b)
            pltpu.sync_copy(buf_a, acc.at[col_v.at[SB - 2]], add=True)
            cp.wait()
            pltpu.sync_copy(buf_b, acc.at[col_v.at[SB - 1]], add=True)
        plsc.subcore_barrier()
        for j in range(RPS // CHUNK):
            off = s * RPS + j * CHUNK
            pltpu.sync_copy(acc.at[pl.ds(off, CHUNK)], buf_a)
            pltpu.sync_copy(buf_a, out_hbm.at[c, pl.ds(off, CHUNK)])

    return k(hs, row2d, col2d)


def _dinv_of(deg_ref):
    d = deg_ref[0, :, 0:1] + deg_ref[1, :, 0:1]
    return lax.rsqrt(1.0 + d)


def _tc_pre(x, W1, deg2):
    def body(x_ref, w_ref, deg_ref, out_ref):
        dinv = _dinv_of(deg_ref)
        out_ref[...] = dinv * jnp.dot(x_ref[...], w_ref[...],
                                      preferred_element_type=f32,
                                      precision=_HIGH)
    return pl.pallas_call(
        body,
        grid=(NBLK,),
        in_specs=[
            pl.BlockSpec((BLK, H), lambda i: (i, 0)),
            pl.BlockSpec((H, H), lambda i: (0, 0)),
            pl.BlockSpec((2, BLK, H), lambda i: (0, i, 0)),
        ],
        out_specs=pl.BlockSpec((BLK, H), lambda i: (i, 0)),
        out_shape=jax.ShapeDtypeStruct((NP, H), f32),
    )(x, W1, deg2)


def _tc_mid(agg2, hs, hprev, deg2, b, Wn, sub):
    def body(agg_ref, hs_ref, hp_ref, deg_ref, b_ref, w_ref, h_ref, hsn_ref):
        dinv = _dinv_of(deg_ref)
        agg = agg_ref[0] + agg_ref[1]
        h = jnp.maximum(dinv * (agg + hs_ref[...]) + b_ref[...], 0.0)
        if sub:
            h = h - hp_ref[...]
        h_ref[...] = h
        hsn_ref[...] = dinv * jnp.dot(h, w_ref[...],
                                      preferred_element_type=f32,
                                      precision=_HIGH)
    return pl.pallas_call(
        body,
        grid=(NBLK,),
        in_specs=[
            pl.BlockSpec((2, BLK, H), lambda i: (0, i, 0)),
            pl.BlockSpec((BLK, H), lambda i: (i, 0)),
            pl.BlockSpec((BLK, H), lambda i: (i, 0)),
            pl.BlockSpec((2, BLK, H), lambda i: (0, i, 0)),
            pl.BlockSpec((1, H), lambda i: (0, 0)),
            pl.BlockSpec((H, H), lambda i: (0, 0)),
        ],
        out_specs=[
            pl.BlockSpec((BLK, H), lambda i: (i, 0)),
            pl.BlockSpec((BLK, H), lambda i: (i, 0)),
        ],
        out_shape=[
            jax.ShapeDtypeStruct((NP, H), f32),
            jax.ShapeDtypeStruct((NP, H), f32),
        ],
    )(agg2, hs, hprev, deg2, b, Wn)


def _tc_final(agg2, hs, hprev, deg2, batch2d, b, Wm1, bm1, Wm2, bm2):
    def body(agg_ref, hs_ref, hp_ref, deg_ref, bt_ref, b_ref,
             wm1_ref, bm1_ref, wm2_ref, bm2_ref, out_ref, sums, cnts):
        i = pl.program_id(0)

        @pl.when(i == 0)
        def _init():
            sums[...] = jnp.zeros((B, H), f32)
            cnts[...] = jnp.zeros((B, 1), f32)

        dinv = _dinv_of(deg_ref)
        agg = agg_ref[0] + agg_ref[1]
        h = jnp.maximum(dinv * (agg + hs_ref[...]) + b_ref[...], 0.0)
        h = h - hp_ref[...]
        bt = bt_ref[...].reshape(1, BLK)
        onehot = (lax.broadcasted_iota(i32, (B, BLK), 0) == bt).astype(f32)
        sums[...] += jnp.dot(onehot, h, preferred_element_type=f32,
                             precision=_HIGH)
        cnts[...] += jnp.sum(onehot, axis=1, keepdims=True)

        @pl.when(i == NBLK - 1)
        def _finish():
            g = sums[...] / jnp.maximum(cnts[...], 1.0)
            g = g / jnp.sqrt(jnp.sum(g * g, axis=1, keepdims=True))
            g = jnp.maximum(jnp.dot(g, wm1_ref[...],
                                    preferred_element_type=f32,
                                    precision=_HIGH) + bm1_ref[...], 0.0)
            g = g / jnp.sqrt(jnp.sum(g * g, axis=1, keepdims=True))
            g = jnp.dot(g, wm2_ref[...], preferred_element_type=f32,
                        precision=_HIGH) + bm2_ref[...]
            g = g / jnp.sqrt(jnp.sum(g * g, axis=1, keepdims=True))
            out_ref[...] = g

    return pl.pallas_call(
        body,
        grid=(NBLK,),
        in_specs=[
            pl.BlockSpec((2, BLK, H), lambda i: (0, i, 0)),
            pl.BlockSpec((BLK, H), lambda i: (i, 0)),
            pl.BlockSpec((BLK, H), lambda i: (i, 0)),
            pl.BlockSpec((2, BLK, H), lambda i: (0, i, 0)),
            pl.BlockSpec((BLK, 1), lambda i: (i, 0)),
            pl.BlockSpec((1, H), lambda i: (0, 0)),
            pl.BlockSpec((H, NHID), lambda i: (0, 0)),
            pl.BlockSpec((1, NHID), lambda i: (0, 0)),
            pl.BlockSpec((NHID, H), lambda i: (0, 0)),
            pl.BlockSpec((1, H), lambda i: (0, 0)),
        ],
        out_specs=pl.BlockSpec((B, H), lambda i: (0, 0)),
        out_shape=jax.ShapeDtypeStruct((B, H), f32),
        scratch_shapes=[
            pltpu.VMEM((B, H), f32),
            pltpu.VMEM((B, 1), f32),
        ],
    )(agg2, hs, hprev, deg2, batch2d, b, Wm1, bm1, Wm2, bm2)


def kernel(x, edge_index, batch, W1, b1, W2, b2, W3, b3, W4, b4, W5, b5,
           Wm1, bm1, Wm2, bm2):
    row = edge_index[0].astype(i32)
    col = edge_index[1].astype(i32)
    fill = jnp.full((EPAD - E,), DUMMY, i32)
    row2d = jnp.concatenate([row, fill]).reshape(NCHUNKS, CHUNK)
    col2d = jnp.concatenate([col, fill]).reshape(NCHUNKS, CHUNK)
    x_pad = jnp.pad(x, ((0, NP - N), (0, 0)))
    batch2d = jnp.pad(batch.astype(i32), (0, NP - N),
                      constant_values=B).reshape(NP, 1)
    b1r, b2r, b3r, b4r, b5r = (v.reshape(1, H) for v in (b1, b2, b3, b4, b5))
    bm1r = bm1.reshape(1, NHID)
    bm2r = bm2.reshape(1, H)

    deg2 = _sc_deg(col2d)
    hs1 = _tc_pre(x_pad, W1, deg2)
    agg1 = _sc_agg(hs1, row2d, col2d)
    h1, hs2 = _tc_mid(agg1, hs1, x_pad, deg2, b1r, W2, sub=False)
    agg2 = _sc_agg(hs2, row2d, col2d)
    h2, hs3 = _tc_mid(agg2, hs2, h1, deg2, b2r, W3, sub=False)
    agg3 = _sc_agg(hs3, row2d, col2d)
    h3, hs4 = _tc_mid(agg3, hs3, h2, deg2, b3r, W4, sub=False)
    agg4 = _sc_agg(hs4, row2d, col2d)
    h4, hs5 = _tc_mid(agg4, hs4, h3, deg2, b4r, W5, sub=True)
    agg5 = _sc_agg(hs5, row2d, col2d)
    return _tc_final(agg5, hs5, h4, deg2, batch2d, b5r, Wm1, bm1r, Wm2, bm2r)


# speedup vs baseline: 18.8736x; 3.0179x over previous
import functools

import jax
import jax.numpy as jnp
from jax import lax
from jax.experimental import pallas as pl
from jax.experimental.pallas import tpu as pltpu
from jax.experimental.pallas import tpu_sc as plsc

N = 10000
NP = 10240
E = 320000
CHUNK = 128
NCHUNKS = 2560
EPAD = NCHUNKS * CHUNK
NSUB = 16
NCORE = 2
CPW = NCHUNKS // (NSUB * NCORE)
SB = 16
H = 128
NHID = 256
B = 256
RPS = NP // NSUB
DUMMY = N + 16
BLK = 1024
NBLK = NP // BLK

f32 = jnp.float32
i32 = jnp.int32
_HIGH = lax.Precision.HIGHEST


def _fill(ref, nrows, width, value):
    def body(i, carry):
        for j in range(width // 16):
            ref[i, pl.ds(j * 16, 16)] = jnp.full((16,), value, f32)
        return carry
    lax.fori_loop(0, nrows, body, 0)


def _sc_deg(col2d):
    mesh = plsc.VectorSubcoreMesh(core_axis_name="c", subcore_axis_name="s")

    @functools.partial(
        pl.kernel,
        out_type=jax.ShapeDtypeStruct((NCORE, NP, H), f32),
        mesh=mesh,
        scratch_types=[
            pltpu.VMEM((CPW, CHUNK), i32),
            pltpu.VMEM((CHUNK, H), f32),
            pltpu.VMEM((CHUNK, H), f32),
            pltpu.VMEM_SHARED((NP, H), f32),
        ],
    )
    def k(col_hbm, out_hbm, col_v, zb, ob, acc):
        c = lax.axis_index("c")
        s = lax.axis_index("s")
        wid = c * NSUB + s
        pltpu.sync_copy(col_hbm.at[pl.ds(wid * CPW, CPW)], col_v)
        _fill(zb, CHUNK, H, 0.0)
        _fill(ob, CHUNK, H, 1.0)
        for j in range(RPS // CHUNK):
            pltpu.sync_copy(zb, acc.at[pl.ds(s * RPS + j * CHUNK, CHUNK)])
        plsc.subcore_barrier()

        def body(i, carry):
            pltpu.sync_copy(ob, acc.at[col_v.at[i]], add=True)
            return carry
        lax.fori_loop(0, CPW, body, 0)
        plsc.subcore_barrier()
        for j in range(RPS // CHUNK):
            off = s * RPS + j * CHUNK
            pltpu.sync_copy(acc.at[pl.ds(off, CHUNK)], zb)
            pltpu.sync_copy(zb, out_hbm.at[c, pl.ds(off, CHUNK)])

    return k(col2d)


def _sc_agg(hs, row2d, col2d):
    mesh = plsc.VectorSubcoreMesh(core_axis_name="c", subcore_axis_name="s")

    @functools.partial(
        pl.kernel,
        out_type=jax.ShapeDtypeStruct((NCORE, NP, H), f32),
        mesh=mesh,
        scratch_types=[
            pltpu.VMEM((SB, CHUNK), i32),
            pltpu.VMEM((SB, CHUNK), i32),
            pltpu.VMEM((CHUNK, H), f32),
            pltpu.VMEM((CHUNK, H), f32),
            pltpu.VMEM_SHARED((NP, H), f32),
            pltpu.SemaphoreType.DMA,
            pltpu.SemaphoreType.DMA,
        ],
    )
    def k(hs_hbm, row_hbm, col_hbm, out_hbm, row_v, col_v, buf_a, buf_b,
          acc, sem_a, sem_b):
        c = lax.axis_index("c")
        s = lax.axis_index("s")
        wid = c * NSUB + s
        _fill(buf_a, CHUNK, H, 0.0)
        for j in range(RPS // CHUNK):
            pltpu.sync_copy(buf_a, acc.at[pl.ds(s * RPS + j * CHUNK, CHUNK)])
        plsc.subcore_barrier()

        for t in range(CPW // SB):
            base = wid * CPW + t * SB
            pltpu.sync_copy(row_hbm.at[pl.ds(base, SB)], row_v)
            pltpu.sync_copy(col_hbm.at[pl.ds(base, SB)], col_v)
            pltpu.async_copy(hs_hbm.at[row_v.at[0]], buf_a, sem_a).wait()

            def body(i, carry):
                cp = pltpu.async_copy(hs_hbm.at[row_v.at[i + 1]], buf_b, sem_b)
                pltpu.sync_copy(buf_a, acc.at[col_v.at[i]], add=True)
                cp.wait()
                cp = pltpu.async_copy(hs_hbm.at[row_v.at[i + 2]], buf_a, sem_a)
                pltpu.sync_copy(buf_b, acc.at[col_v.at[i + 1]], add=True)
                cp.wait()
                return carry
            lax.fori_loop(0, SB // 2 - 1, lambda u, cr: body(2 * u, cr), 0)
            cp = pltpu.async_copy(hs_hbm.at[row_v.at[SB - 1]], buf_b, sem_b)
            pltpu.sync_copy(buf_a, acc.at[col_v.at[SB - 2]], add=True)
            cp.wait()
            pltpu.sync_copy(buf_b, acc.at[col_v.at[SB - 1]], add=True)
        plsc.subcore_barrier()
        for j in range(RPS // CHUNK):
            off = s * RPS + j * CHUNK
            pltpu.sync_copy(acc.at[pl.ds(off, CHUNK)], buf_a)
            pltpu.sync_copy(buf_a, out_hbm.at[c, pl.ds(off, CHUNK)])

    return k(hs, row2d, col2d)


def _dinv_of(deg_ref):
    d = deg_ref[0, :, 0:1] + deg_ref[1, :, 0:1]
    return lax.rsqrt(1.0 + d)


def _tc_pre(x, W1, deg2):
    def body(x_ref, w_ref, deg_ref, out_ref):
        dinv = _dinv_of(deg_ref)
        out_ref[...] = dinv * jnp.dot(x_ref[...], w_ref[...],
                                      preferred_element_type=f32,
                                      precision=_HIGH)
    return pl.pallas_call(
        body,
        grid=(NBLK,),
        in_specs=[
            pl.BlockSpec((BLK, H), lambda i: (i, 0)),
            pl.BlockSpec((H, H), lambda i: (0, 0)),
            pl.BlockSpec((2, BLK, H), lambda i: (0, i, 0)),
        ],
        out_specs=pl.BlockSpec((BLK, H), lambda i: (i, 0)),
        out_shape=jax.ShapeDtypeStruct((NP, H), f32),
    )(x, W1, deg2)


def _tc_mid(agg2, hs, hprev, deg2, b, Wn, sub):
    def body(agg_ref, hs_ref, hp_ref, deg_ref, b_ref, w_ref, h_ref, hsn_ref):
        dinv = _dinv_of(deg_ref)
        agg = agg_ref[0] + agg_ref[1]
        h = jnp.maximum(dinv * (agg + hs_ref[...]) + b_ref[...], 0.0)
        if sub:
            h = h - hp_ref[...]
        h_ref[...] = h
        hsn_ref[...] = dinv * jnp.dot(h, w_ref[...],
                                      preferred_element_type=f32,
                                      precision=_HIGH)
    return pl.pallas_call(
        body,
        grid=(NBLK,),
        in_specs=[
            pl.BlockSpec((2, BLK, H), lambda i: (0, i, 0)),
            pl.BlockSpec((BLK, H), lambda i: (i, 0)),
            pl.BlockSpec((BLK, H), lambda i: (i, 0)),
            pl.BlockSpec((2, BLK, H), lambda i: (0, i, 0)),
            pl.BlockSpec((1, H), lambda i: (0, 0)),
            pl.BlockSpec((H, H), lambda i: (0, 0)),
        ],
        out_specs=[
            pl.BlockSpec((BLK, H), lambda i: (i, 0)),
            pl.BlockSpec((BLK, H), lambda i: (i, 0)),
        ],
        out_shape=[
            jax.ShapeDtypeStruct((NP, H), f32),
            jax.ShapeDtypeStruct((NP, H), f32),
        ],
    )(agg2, hs, hprev, deg2, b, Wn)


def _tc_final(agg2, hs, hprev, deg2, batch2d, b, Wm1, bm1, Wm2, bm2):
    def body(agg_ref, hs_ref, hp_ref, deg_ref, bt_ref, b_ref,
             wm1_ref, bm1_ref, wm2_ref, bm2_ref, out_ref, sums, cnts):
        i = pl.program_id(0)

        @pl.when(i == 0)
        def _init():
            sums[...] = jnp.zeros((B, H), f32)
            cnts[...] = jnp.zeros((B, 1), f32)

        dinv = _dinv_of(deg_ref)
        agg = agg_ref[0] + agg_ref[1]
        h = jnp.maximum(dinv * (agg + hs_ref[...]) + b_ref[...], 0.0)
        h = h - hp_ref[...]
        bt = bt_ref[...].reshape(1, BLK)
        onehot = (lax.broadcasted_iota(i32, (B, BLK), 0) == bt).astype(f32)
        sums[...] += jnp.dot(onehot, h, preferred_element_type=f32,
                             precision=_HIGH)
        cnts[...] += jnp.sum(onehot, axis=1, keepdims=True)

        @pl.when(i == NBLK - 1)
        def _finish():
            g = sums[...] / jnp.maximum(cnts[...], 1.0)
            g = g / jnp.sqrt(jnp.sum(g * g, axis=1, keepdims=True))
            g = jnp.maximum(jnp.dot(g, wm1_ref[...],
                                    preferred_element_type=f32,
                                    precision=_HIGH) + bm1_ref[...], 0.0)
            g = g / jnp.sqrt(jnp.sum(g * g, axis=1, keepdims=True))
            g = jnp.dot(g, wm2_ref[...], preferred_element_type=f32,
                        precision=_HIGH) + bm2_ref[...]
            g = g / jnp.sqrt(jnp.sum(g * g, axis=1, keepdims=True))
            out_ref[...] = g

    return pl.pallas_call(
        body,
        grid=(NBLK,),
        in_specs=[
            pl.BlockSpec((2, BLK, H), lambda i: (0, i, 0)),
            pl.BlockSpec((BLK, H), lambda i: (i, 0)),
            pl.BlockSpec((BLK, H), lambda i: (i, 0)),
            pl.BlockSpec((2, BLK, H), lambda i: (0, i, 0)),
            pl.BlockSpec((BLK, 1), lambda i: (i, 0)),
            pl.BlockSpec((1, H), lambda i: (0, 0)),
            pl.BlockSpec((H, NHID), lambda i: (0, 0)),
            pl.BlockSpec((1, NHID), lambda i: (0, 0)),
            pl.BlockSpec((NHID, H), lambda i: (0, 0)),
            pl.BlockSpec((1, H), lambda i: (0, 0)),
        ],
        out_specs=pl.BlockSpec((B, H), lambda i: (0, 0)),
        out_shape=jax.ShapeDtypeStruct((B, H), f32),
        scratch_shapes=[
            pltpu.VMEM((B, H), f32),
            pltpu.VMEM((B, 1), f32),
        ],
    )(agg2, hs, hprev, deg2, batch2d, b, Wm1, bm1, Wm2, bm2)


def kernel(x, edge_index, batch, W1, b1, W2, b2, W3, b3, W4, b4, W5, b5,
           Wm1, bm1, Wm2, bm2):
    row = edge_index[0].astype(i32)
    col = edge_index[1].astype(i32)
    fill = N + jnp.arange(EPAD - E, dtype=i32) % (NP - N)
    row2d = jnp.concatenate([row, fill]).reshape(NCHUNKS, CHUNK)
    col2d = jnp.concatenate([col, fill]).reshape(NCHUNKS, CHUNK)
    x_pad = jnp.pad(x, ((0, NP - N), (0, 0)))
    batch2d = jnp.pad(batch.astype(i32), (0, NP - N),
                      constant_values=B).reshape(NP, 1)
    b1r, b2r, b3r, b4r, b5r = (v.reshape(1, H) for v in (b1, b2, b3, b4, b5))
    bm1r = bm1.reshape(1, NHID)
    bm2r = bm2.reshape(1, H)

    deg2 = _sc_deg(col2d)
    hs1 = _tc_pre(x_pad, W1, deg2)
    agg1 = _sc_agg(hs1, row2d, col2d)
    h1, hs2 = _tc_mid(agg1, hs1, x_pad, deg2, b1r, W2, sub=False)
    agg2 = _sc_agg(hs2, row2d, col2d)
    h2, hs3 = _tc_mid(agg2, hs2, h1, deg2, b2r, W3, sub=False)
    agg3 = _sc_agg(hs3, row2d, col2d)
    h3, hs4 = _tc_mid(agg3, hs3, h2, deg2, b3r, W4, sub=False)
    agg4 = _sc_agg(hs4, row2d, col2d)
    h4, hs5 = _tc_mid(agg4, hs4, h3, deg2, b4r, W5, sub=True)
    agg5 = _sc_agg(hs5, row2d, col2d)
    return _tc_final(agg5, hs5, h4, deg2, batch2d, b5r, Wm1, bm1r, Wm2, bm2r)


# speedup vs baseline: 19.5614x; 1.0364x over previous
import functools

import jax
import jax.numpy as jnp
from jax import lax
from jax.experimental import pallas as pl
from jax.experimental.pallas import tpu as pltpu
from jax.experimental.pallas import tpu_sc as plsc

N = 10000
NP = 10240
E = 320000
CHUNK = 128
NCHUNKS = 2560
EPAD = NCHUNKS * CHUNK
NSUB = 16
NCORE = 2
CPW = NCHUNKS // (NSUB * NCORE)
SB = 40
H = 128
NHID = 256
B = 256
RPS = NP // NSUB
DUMMY = N + 16
BLK = 1024
NBLK = NP // BLK

f32 = jnp.float32
i32 = jnp.int32
_HIGH = lax.Precision.HIGHEST


def _fill(ref, nrows, width, value):
    def body(i, carry):
        for j in range(width // 16):
            ref[i, pl.ds(j * 16, 16)] = jnp.full((16,), value, f32)
        return carry
    lax.fori_loop(0, nrows, body, 0)


def _sc_deg(col2d):
    mesh = plsc.VectorSubcoreMesh(core_axis_name="c", subcore_axis_name="s")

    @functools.partial(
        pl.kernel,
        out_type=jax.ShapeDtypeStruct((NCORE, NP, H), f32),
        mesh=mesh,
        scratch_types=[
            pltpu.VMEM((CPW, CHUNK), i32),
            pltpu.VMEM((CHUNK, H), f32),
            pltpu.VMEM((CHUNK, H), f32),
            pltpu.VMEM_SHARED((NP, H), f32),
        ],
    )
    def k(col_hbm, out_hbm, col_v, zb, ob, acc):
        c = lax.axis_index("c")
        s = lax.axis_index("s")
        wid = c * NSUB + s
        pltpu.sync_copy(col_hbm.at[pl.ds(wid * CPW, CPW)], col_v)
        _fill(zb, CHUNK, H, 0.0)
        _fill(ob, CHUNK, H, 1.0)
        for j in range(RPS // CHUNK):
            pltpu.sync_copy(zb, acc.at[pl.ds(s * RPS + j * CHUNK, CHUNK)])
        plsc.subcore_barrier()

        def body(i, carry):
            pltpu.sync_copy(ob, acc.at[col_v.at[i]], add=True)
            return carry
        lax.fori_loop(0, CPW, body, 0)
        plsc.subcore_barrier()
        for j in range(RPS // CHUNK):
            off = s * RPS + j * CHUNK
            pltpu.sync_copy(acc.at[pl.ds(off, CHUNK)], zb)
            pltpu.sync_copy(zb, out_hbm.at[c, pl.ds(off, CHUNK)])

    return k(col2d)


def _sc_agg(hs, row2d, col2d):
    mesh = plsc.VectorSubcoreMesh(core_axis_name="c", subcore_axis_name="s")

    @functools.partial(
        pl.kernel,
        out_type=jax.ShapeDtypeStruct((NCORE, NP, H), f32),
        mesh=mesh,
        scratch_types=[
            pltpu.VMEM((SB, CHUNK), i32),
            pltpu.VMEM((SB, CHUNK), i32),
            pltpu.VMEM((CHUNK, H), f32),
            pltpu.VMEM((CHUNK, H), f32),
            pltpu.VMEM_SHARED((NP, H), f32),
            pltpu.SemaphoreType.DMA,
            pltpu.SemaphoreType.DMA,
        ],
    )
    def k(hs_hbm, row_hbm, col_hbm, out_hbm, row_v, col_v, buf_a, buf_b,
          acc, sem_a, sem_b):
        c = lax.axis_index("c")
        s = lax.axis_index("s")
        wid = c * NSUB + s
        _fill(buf_a, CHUNK, H, 0.0)
        for j in range(RPS // CHUNK):
            pltpu.sync_copy(buf_a, acc.at[pl.ds(s * RPS + j * CHUNK, CHUNK)])
        plsc.subcore_barrier()

        for t in range(CPW // SB):
            base = wid * CPW + t * SB
            pltpu.sync_copy(row_hbm.at[pl.ds(base, SB)], row_v)
            pltpu.sync_copy(col_hbm.at[pl.ds(base, SB)], col_v)
            pltpu.async_copy(hs_hbm.at[row_v.at[0]], buf_a, sem_a).wait()

            def body(i, carry):
                cp = pltpu.async_copy(hs_hbm.at[row_v.at[i + 1]], buf_b, sem_b)
                pltpu.sync_copy(buf_a, acc.at[col_v.at[i]], add=True)
                cp.wait()
                cp = pltpu.async_copy(hs_hbm.at[row_v.at[i + 2]], buf_a, sem_a)
                pltpu.sync_copy(buf_b, acc.at[col_v.at[i + 1]], add=True)
                cp.wait()
                return carry
            lax.fori_loop(0, SB // 2 - 1, lambda u, cr: body(2 * u, cr), 0)
            cp = pltpu.async_copy(hs_hbm.at[row_v.at[SB - 1]], buf_b, sem_b)
            pltpu.sync_copy(buf_a, acc.at[col_v.at[SB - 2]], add=True)
            cp.wait()
            pltpu.sync_copy(buf_b, acc.at[col_v.at[SB - 1]], add=True)
        plsc.subcore_barrier()
        for j in range(RPS // CHUNK):
            off = s * RPS + j * CHUNK
            pltpu.sync_copy(acc.at[pl.ds(off, CHUNK)], buf_a)
            pltpu.sync_copy(buf_a, out_hbm.at[c, pl.ds(off, CHUNK)])

    return k(hs, row2d, col2d)


def _dinv_of(deg_ref):
    d = deg_ref[0, :, 0:1] + deg_ref[1, :, 0:1]
    return lax.rsqrt(1.0 + d)


def _tc_pre(x, W1, deg2):
    def body(x_ref, w_ref, deg_ref, out_ref):
        dinv = _dinv_of(deg_ref)
        out_ref[...] = dinv * jnp.dot(x_ref[...], w_ref[...],
                                      preferred_element_type=f32,
                                      precision=_HIGH)
    return pl.pallas_call(
        body,
        grid=(NBLK,),
        in_specs=[
            pl.BlockSpec((BLK, H), lambda i: (i, 0)),
            pl.BlockSpec((H, H), lambda i: (0, 0)),
            pl.BlockSpec((2, BLK, H), lambda i: (0, i, 0)),
        ],
        out_specs=pl.BlockSpec((BLK, H), lambda i: (i, 0)),
        out_shape=jax.ShapeDtypeStruct((NP, H), f32),
    )(x, W1, deg2)


def _tc_mid(agg2, hs, hprev, deg2, b, Wn, sub):
    def body(agg_ref, hs_ref, hp_ref, deg_ref, b_ref, w_ref, h_ref, hsn_ref):
        dinv = _dinv_of(deg_ref)
        agg = agg_ref[0] + agg_ref[1]
        h = jnp.maximum(dinv * (agg + hs_ref[...]) + b_ref[...], 0.0)
        if sub:
            h = h - hp_ref[...]
        h_ref[...] = h
        hsn_ref[...] = dinv * jnp.dot(h, w_ref[...],
                                      preferred_element_type=f32,
                                      precision=_HIGH)
    return pl.pallas_call(
        body,
        grid=(NBLK,),
        in_specs=[
            pl.BlockSpec((2, BLK, H), lambda i: (0, i, 0)),
            pl.BlockSpec((BLK, H), lambda i: (i, 0)),
            pl.BlockSpec((BLK, H), lambda i: (i, 0)),
            pl.BlockSpec((2, BLK, H), lambda i: (0, i, 0)),
            pl.BlockSpec((1, H), lambda i: (0, 0)),
            pl.BlockSpec((H, H), lambda i: (0, 0)),
        ],
        out_specs=[
            pl.BlockSpec((BLK, H), lambda i: (i, 0)),
            pl.BlockSpec((BLK, H), lambda i: (i, 0)),
        ],
        out_shape=[
            jax.ShapeDtypeStruct((NP, H), f32),
            jax.ShapeDtypeStruct((NP, H), f32),
        ],
    )(agg2, hs, hprev, deg2, b, Wn)


def _tc_final(agg2, hs, hprev, deg2, batch2d, b, Wm1, bm1, Wm2, bm2):
    def body(agg_ref, hs_ref, hp_ref, deg_ref, bt_ref, b_ref,
             wm1_ref, bm1_ref, wm2_ref, bm2_ref, out_ref, sums, cnts):
        i = pl.program_id(0)

        @pl.when(i == 0)
        def _init():
            sums[...] = jnp.zeros((B, H), f32)
            cnts[...] = jnp.zeros((B, 1), f32)

        dinv = _dinv_of(deg_ref)
        agg = agg_ref[0] + agg_ref[1]
        h = jnp.maximum(dinv * (agg + hs_ref[...]) + b_ref[...], 0.0)
        h = h - hp_ref[...]
        bt = bt_ref[...].reshape(1, BLK)
        onehot = (lax.broadcasted_iota(i32, (B, BLK), 0) == bt).astype(f32)
        sums[...] += jnp.dot(onehot, h, preferred_element_type=f32,
                             precision=_HIGH)
        cnts[...] += jnp.sum(onehot, axis=1, keepdims=True)

        @pl.when(i == NBLK - 1)
        def _finish():
            g = sums[...] / jnp.maximum(cnts[...], 1.0)
            g = g / jnp.sqrt(jnp.sum(g * g, axis=1, keepdims=True))
            g = jnp.maximum(jnp.dot(g, wm1_ref[...],
                                    preferred_element_type=f32,
                                    precision=_HIGH) + bm1_ref[...], 0.0)
            g = g / jnp.sqrt(jnp.sum(g * g, axis=1, keepdims=True))
            g = jnp.dot(g, wm2_ref[...], preferred_element_type=f32,
                        precision=_HIGH) + bm2_ref[...]
            g = g / jnp.sqrt(jnp.sum(g * g, axis=1, keepdims=True))
            out_ref[...] = g

    return pl.pallas_call(
        body,
        grid=(NBLK,),
        in_specs=[
            pl.BlockSpec((2, BLK, H), lambda i: (0, i, 0)),
            pl.BlockSpec((BLK, H), lambda i: (i, 0)),
            pl.BlockSpec((BLK, H), lambda i: (i, 0)),
            pl.BlockSpec((2, BLK, H), lambda i: (0, i, 0)),
            pl.BlockSpec((BLK, 1), lambda i: (i, 0)),
            pl.BlockSpec((1, H), lambda i: (0, 0)),
            pl.BlockSpec((H, NHID), lambda i: (0, 0)),
            pl.BlockSpec((1, NHID), lambda i: (0, 0)),
            pl.BlockSpec((NHID, H), lambda i: (0, 0)),
            pl.BlockSpec((1, H), lambda i: (0, 0)),
        ],
        out_specs=pl.BlockSpec((B, H), lambda i: (0, 0)),
        out_shape=jax.ShapeDtypeStruct((B, H), f32),
        scratch_shapes=[
            pltpu.VMEM((B, H), f32),
            pltpu.VMEM((B, 1), f32),
        ],
    )(agg2, hs, hprev, deg2, batch2d, b, Wm1, bm1, Wm2, bm2)


def kernel(x, edge_index, batch, W1, b1, W2, b2, W3, b3, W4, b4, W5, b5,
           Wm1, bm1, Wm2, bm2):
    row = edge_index[0].astype(i32)
    col = edge_index[1].astype(i32)
    fill = N + jnp.arange(EPAD - E, dtype=i32) % (NP - N)
    row2d = jnp.concatenate([row, fill]).reshape(NCHUNKS, CHUNK)
    col2d = jnp.concatenate([col, fill]).reshape(NCHUNKS, CHUNK)
    x_pad = jnp.pad(x, ((0, NP - N), (0, 0)))
    batch2d = jnp.pad(batch.astype(i32), (0, NP - N),
                      constant_values=B).reshape(NP, 1)
    b1r, b2r, b3r, b4r, b5r = (v.reshape(1, H) for v in (b1, b2, b3, b4, b5))
    bm1r = bm1.reshape(1, NHID)
    bm2r = bm2.reshape(1, H)

    deg2 = _sc_deg(col2d)
    hs1 = _tc_pre(x_pad, W1, deg2)
    agg1 = _sc_agg(hs1, row2d, col2d)
    h1, hs2 = _tc_mid(agg1, hs1, x_pad, deg2, b1r, W2, sub=False)
    agg2 = _sc_agg(hs2, row2d, col2d)
    h2, hs3 = _tc_mid(agg2, hs2, h1, deg2, b2r, W3, sub=False)
    agg3 = _sc_agg(hs3, row2d, col2d)
    h3, hs4 = _tc_mid(agg3, hs3, h2, deg2, b3r, W4, sub=False)
    agg4 = _sc_agg(hs4, row2d, col2d)
    h4, hs5 = _tc_mid(agg4, hs4, h3, deg2, b4r, W5, sub=True)
    agg5 = _sc_agg(hs5, row2d, col2d)
    return _tc_final(agg5, hs5, h4, deg2, batch2d, b5r, Wm1, bm1r, Wm2, bm2r)


# speedup vs baseline: 19.5893x; 1.0014x over previous
import functools

import jax
import jax.numpy as jnp
from jax import lax
from jax.experimental import pallas as pl
from jax.experimental.pallas import tpu as pltpu
from jax.experimental.pallas import tpu_sc as plsc

N = 10000
NP = 10240
E = 320000
CHUNK = 128
NCHUNKS = 2560
EPAD = NCHUNKS * CHUNK
NSUB = 16
NCORE = 2
CPW = NCHUNKS // (NSUB * NCORE)
SB = 40
H = 128
NHID = 256
B = 256
RPS = NP // NSUB
DUMMY = N + 16
BLK = 1024
NBLK = NP // BLK

f32 = jnp.float32
i32 = jnp.int32
_HIGH = lax.Precision.HIGHEST


def _fill(ref, nrows, width, value):
    def body(i, carry):
        for j in range(width // 16):
            ref[i, pl.ds(j * 16, 16)] = jnp.full((16,), value, f32)
        return carry
    lax.fori_loop(0, nrows, body, 0)


def _sc_deg(col2d):
    mesh = plsc.VectorSubcoreMesh(core_axis_name="c", subcore_axis_name="s")

    @functools.partial(
        pl.kernel,
        out_type=jax.ShapeDtypeStruct((NCORE, NP, H), f32),
        mesh=mesh,
        scratch_types=[
            pltpu.VMEM((CPW, CHUNK), i32),
            pltpu.VMEM((CHUNK, H), f32),
            pltpu.VMEM((CHUNK, H), f32),
            pltpu.VMEM_SHARED((NP, H), f32),
        ],
    )
    def k(col_hbm, out_hbm, col_v, zb, ob, acc):
        c = lax.axis_index("c")
        s = lax.axis_index("s")
        wid = c * NSUB + s
        pltpu.sync_copy(col_hbm.at[pl.ds(wid * CPW, CPW)], col_v)
        _fill(zb, CHUNK, H, 0.0)
        _fill(ob, CHUNK, H, 1.0)
        for j in range(RPS // CHUNK):
            pltpu.sync_copy(zb, acc.at[pl.ds(s * RPS + j * CHUNK, CHUNK)])
        plsc.subcore_barrier()

        def body(i, carry):
            pltpu.sync_copy(ob, acc.at[col_v.at[i]], add=True)
            return carry
        lax.fori_loop(0, CPW, body, 0)
        plsc.subcore_barrier()
        for j in range(RPS // CHUNK):
            off = s * RPS + j * CHUNK
            pltpu.sync_copy(acc.at[pl.ds(off, CHUNK)], zb)
            pltpu.sync_copy(zb, out_hbm.at[c, pl.ds(off, CHUNK)])

    return k(col2d)


def _sc_agg(hs, row2d, col2d):
    mesh = plsc.VectorSubcoreMesh(core_axis_name="c", subcore_axis_name="s")

    @functools.partial(
        pl.kernel,
        out_type=jax.ShapeDtypeStruct((NCORE, NP, H), f32),
        mesh=mesh,
        scratch_types=[
            pltpu.VMEM((SB, CHUNK), i32),
            pltpu.VMEM((SB, CHUNK), i32),
            pltpu.VMEM((CHUNK, H), f32),
            pltpu.VMEM((CHUNK, H), f32),
            pltpu.VMEM_SHARED((NP, H), f32),
            pltpu.SemaphoreType.DMA,
            pltpu.SemaphoreType.DMA,
        ],
    )
    def k(hs_hbm, row_hbm, col_hbm, out_hbm, row_v, col_v, buf_a, buf_b,
          acc, sem_a, sem_b):
        c = lax.axis_index("c")
        s = lax.axis_index("s")
        wid = c * NSUB + s
        _fill(buf_a, CHUNK, H, 0.0)
        for j in range(RPS // CHUNK):
            pltpu.sync_copy(buf_a, acc.at[pl.ds(s * RPS + j * CHUNK, CHUNK)])
        plsc.subcore_barrier()

        for t in range(CPW // SB):
            base = wid * CPW + t * SB
            pltpu.sync_copy(row_hbm.at[pl.ds(base, SB)], row_v)
            pltpu.sync_copy(col_hbm.at[pl.ds(base, SB)], col_v)
            pltpu.async_copy(hs_hbm.at[row_v.at[0]], buf_a, sem_a).wait()

            def body(i, carry):
                cp = pltpu.async_copy(hs_hbm.at[row_v.at[i + 1]], buf_b, sem_b)
                pltpu.sync_copy(buf_a, acc.at[col_v.at[i]], add=True)
                cp.wait()
                cp = pltpu.async_copy(hs_hbm.at[row_v.at[i + 2]], buf_a, sem_a)
                pltpu.sync_copy(buf_b, acc.at[col_v.at[i + 1]], add=True)
                cp.wait()
                return carry
            lax.fori_loop(0, SB // 2 - 1, lambda u, cr: body(2 * u, cr), 0)
            cp = pltpu.async_copy(hs_hbm.at[row_v.at[SB - 1]], buf_b, sem_b)
            pltpu.sync_copy(buf_a, acc.at[col_v.at[SB - 2]], add=True)
            cp.wait()
            pltpu.sync_copy(buf_b, acc.at[col_v.at[SB - 1]], add=True)
        plsc.subcore_barrier()
        for j in range(RPS // CHUNK):
            off = s * RPS + j * CHUNK
            pltpu.sync_copy(acc.at[pl.ds(off, CHUNK)], buf_a)
            pltpu.sync_copy(buf_a, out_hbm.at[c, pl.ds(off, CHUNK)])

    return k(hs, row2d, col2d)


def _dinv_of(deg_ref):
    d = deg_ref[0, :, 0:1] + deg_ref[1, :, 0:1]
    return lax.rsqrt(1.0 + d)


def _tc_z1(x, W1):
    def body(x_ref, w_ref, out_ref):
        out_ref[...] = jnp.dot(x_ref[...], w_ref[...],
                               preferred_element_type=f32,
                               precision=_HIGH)
    return pl.pallas_call(
        body,
        grid=(NBLK,),
        in_specs=[
            pl.BlockSpec((BLK, H), lambda i: (i, 0)),
            pl.BlockSpec((H, H), lambda i: (0, 0)),
        ],
        out_specs=pl.BlockSpec((BLK, H), lambda i: (i, 0)),
        out_shape=jax.ShapeDtypeStruct((NP, H), f32),
    )(x, W1)


def _tc_scale(z1, deg2):
    def body(z_ref, deg_ref, out_ref):
        out_ref[...] = _dinv_of(deg_ref) * z_ref[...]
    return pl.pallas_call(
        body,
        grid=(NBLK,),
        in_specs=[
            pl.BlockSpec((BLK, H), lambda i: (i, 0)),
            pl.BlockSpec((2, BLK, H), lambda i: (0, i, 0)),
        ],
        out_specs=pl.BlockSpec((BLK, H), lambda i: (i, 0)),
        out_shape=jax.ShapeDtypeStruct((NP, H), f32),
    )(z1, deg2)


def _tc_mid(agg2, hs, hprev, deg2, b, Wn, sub):
    def body(agg_ref, hs_ref, hp_ref, deg_ref, b_ref, w_ref, h_ref, hsn_ref):
        dinv = _dinv_of(deg_ref)
        agg = agg_ref[0] + agg_ref[1]
        h = jnp.maximum(dinv * (agg + hs_ref[...]) + b_ref[...], 0.0)
        if sub:
            h = h - hp_ref[...]
        h_ref[...] = h
        hsn_ref[...] = dinv * jnp.dot(h, w_ref[...],
                                      preferred_element_type=f32,
                                      precision=_HIGH)
    return pl.pallas_call(
        body,
        grid=(NBLK,),
        in_specs=[
            pl.BlockSpec((2, BLK, H), lambda i: (0, i, 0)),
            pl.BlockSpec((BLK, H), lambda i: (i, 0)),
            pl.BlockSpec((BLK, H), lambda i: (i, 0)),
            pl.BlockSpec((2, BLK, H), lambda i: (0, i, 0)),
            pl.BlockSpec((1, H), lambda i: (0, 0)),
            pl.BlockSpec((H, H), lambda i: (0, 0)),
        ],
        out_specs=[
            pl.BlockSpec((BLK, H), lambda i: (i, 0)),
            pl.BlockSpec((BLK, H), lambda i: (i, 0)),
        ],
        out_shape=[
            jax.ShapeDtypeStruct((NP, H), f32),
            jax.ShapeDtypeStruct((NP, H), f32),
        ],
    )(agg2, hs, hprev, deg2, b, Wn)


def _tc_final(agg2, hs, hprev, deg2, batch2d, b, Wm1, bm1, Wm2, bm2):
    def body(agg_ref, hs_ref, hp_ref, deg_ref, bt_ref, b_ref,
             wm1_ref, bm1_ref, wm2_ref, bm2_ref, out_ref, sums, cnts):
        i = pl.program_id(0)

        @pl.when(i == 0)
        def _init():
            sums[...] = jnp.zeros((B, H), f32)
            cnts[...] = jnp.zeros((B, 1), f32)

        dinv = _dinv_of(deg_ref)
        agg = agg_ref[0] + agg_ref[1]
        h = jnp.maximum(dinv * (agg + hs_ref[...]) + b_ref[...], 0.0)
        h = h - hp_ref[...]
        bt = bt_ref[...].reshape(1, BLK)
        onehot = (lax.broadcasted_iota(i32, (B, BLK), 0) == bt).astype(f32)
        sums[...] += jnp.dot(onehot, h, preferred_element_type=f32,
                             precision=_HIGH)
        cnts[...] += jnp.sum(onehot, axis=1, keepdims=True)

        @pl.when(i == NBLK - 1)
        def _finish():
            g = sums[...] / jnp.maximum(cnts[...], 1.0)
            g = g / jnp.sqrt(jnp.sum(g * g, axis=1, keepdims=True))
            g = jnp.maximum(jnp.dot(g, wm1_ref[...],
                                    preferred_element_type=f32,
                                    precision=_HIGH) + bm1_ref[...], 0.0)
            g = g / jnp.sqrt(jnp.sum(g * g, axis=1, keepdims=True))
            g = jnp.dot(g, wm2_ref[...], preferred_element_type=f32,
                        precision=_HIGH) + bm2_ref[...]
            g = g / jnp.sqrt(jnp.sum(g * g, axis=1, keepdims=True))
            out_ref[...] = g

    return pl.pallas_call(
        body,
        grid=(NBLK,),
        in_specs=[
            pl.BlockSpec((2, BLK, H), lambda i: (0, i, 0)),
            pl.BlockSpec((BLK, H), lambda i: (i, 0)),
            pl.BlockSpec((BLK, H), lambda i: (i, 0)),
            pl.BlockSpec((2, BLK, H), lambda i: (0, i, 0)),
            pl.BlockSpec((BLK, 1), lambda i: (i, 0)),
            pl.BlockSpec((1, H), lambda i: (0, 0)),
            pl.BlockSpec((H, NHID), lambda i: (0, 0)),
            pl.BlockSpec((1, NHID), lambda i: (0, 0)),
            pl.BlockSpec((NHID, H), lambda i: (0, 0)),
            pl.BlockSpec((1, H), lambda i: (0, 0)),
        ],
        out_specs=pl.BlockSpec((B, H), lambda i: (0, 0)),
        out_shape=jax.ShapeDtypeStruct((B, H), f32),
        scratch_shapes=[
            pltpu.VMEM((B, H), f32),
            pltpu.VMEM((B, 1), f32),
        ],
    )(agg2, hs, hprev, deg2, batch2d, b, Wm1, bm1, Wm2, bm2)


def kernel(x, edge_index, batch, W1, b1, W2, b2, W3, b3, W4, b4, W5, b5,
           Wm1, bm1, Wm2, bm2):
    row = edge_index[0].astype(i32)
    col = edge_index[1].astype(i32)
    fill = N + jnp.arange(EPAD - E, dtype=i32) % (NP - N)
    row2d = jnp.concatenate([row, fill]).reshape(NCHUNKS, CHUNK)
    col2d = jnp.concatenate([col, fill]).reshape(NCHUNKS, CHUNK)
    x_pad = jnp.pad(x, ((0, NP - N), (0, 0)))
    batch2d = jnp.pad(batch.astype(i32), (0, NP - N),
                      constant_values=B).reshape(NP, 1)
    b1r, b2r, b3r, b4r, b5r = (v.reshape(1, H) for v in (b1, b2, b3, b4, b5))
    bm1r = bm1.reshape(1, NHID)
    bm2r = bm2.reshape(1, H)

    z1 = _tc_z1(x_pad, W1)
    deg2 = _sc_deg(col2d)
    hs1 = _tc_scale(z1, deg2)
    agg1 = _sc_agg(hs1, row2d, col2d)
    h1, hs2 = _tc_mid(agg1, hs1, x_pad, deg2, b1r, W2, sub=False)
    agg2 = _sc_agg(hs2, row2d, col2d)
    h2, hs3 = _tc_mid(agg2, hs2, h1, deg2, b2r, W3, sub=False)
    agg3 = _sc_agg(hs3, row2d, col2d)
    h3, hs4 = _tc_mid(agg3, hs3, h2, deg2, b3r, W4, sub=False)
    agg4 = _sc_agg(hs4, row2d, col2d)
    h4, hs5 = _tc_mid(agg4, hs4, h3, deg2, b4r, W5, sub=True)
    agg5 = _sc_agg(hs5, row2d, col2d)
    return _tc_final(agg5, hs5, h4, deg2, batch2d, b5r, Wm1, bm1r, Wm2, bm2r)


# speedup vs baseline: 19.8847x; 1.0151x over previous
import functools

import jax
import jax.numpy as jnp
from jax import lax
from jax.experimental import pallas as pl
from jax.experimental.pallas import tpu as pltpu
from jax.experimental.pallas import tpu_sc as plsc

N = 10000
NP = 10240
E = 320000
CHUNK = 128
NCHUNKS = 2560
EPAD = NCHUNKS * CHUNK
NSUB = 16
NCORE = 2
CPW = NCHUNKS // (NSUB * NCORE)
SB = 40
H = 128
NHID = 256
B = 256
RPS = NP // NSUB
DUMMY = N + 16
BLK = 1024
NBLK = NP // BLK

f32 = jnp.float32
i32 = jnp.int32
_HIGH = lax.Precision.HIGHEST


def _fill(ref, nrows, width, value):
    def body(i, carry):
        for j in range(width // 16):
            ref[i, pl.ds(j * 16, 16)] = jnp.full((16,), value, f32)
        return carry
    lax.fori_loop(0, nrows, body, 0)


def _sc_deg(col2d):
    mesh = plsc.VectorSubcoreMesh(core_axis_name="c", subcore_axis_name="s")

    @functools.partial(
        pl.kernel,
        out_type=jax.ShapeDtypeStruct((NCORE, NP, H), f32),
        mesh=mesh,
        scratch_types=[
            pltpu.VMEM((CPW, CHUNK), i32),
            pltpu.VMEM((CHUNK, H), f32),
            pltpu.VMEM((CHUNK, H), f32),
            pltpu.VMEM_SHARED((NP, H), f32),
        ],
    )
    def k(col_hbm, out_hbm, col_v, zb, ob, acc):
        c = lax.axis_index("c")
        s = lax.axis_index("s")
        wid = c * NSUB + s
        pltpu.sync_copy(col_hbm.at[pl.ds(wid * CPW, CPW)], col_v)
        _fill(zb, CHUNK, H, 0.0)
        _fill(ob, CHUNK, H, 1.0)
        for j in range(RPS // CHUNK):
            pltpu.sync_copy(zb, acc.at[pl.ds(s * RPS + j * CHUNK, CHUNK)])
        plsc.subcore_barrier()

        def body(i, carry):
            pltpu.sync_copy(ob, acc.at[col_v.at[i]], add=True)
            return carry
        lax.fori_loop(0, CPW, body, 0)
        plsc.subcore_barrier()
        for j in range(RPS // CHUNK):
            off = s * RPS + j * CHUNK
            pltpu.sync_copy(acc.at[pl.ds(off, CHUNK)], zb)
            pltpu.sync_copy(zb, out_hbm.at[c, pl.ds(off, CHUNK)])

    return k(col2d)


def _sc_agg(hs, row2d, col2d):
    mesh = plsc.VectorSubcoreMesh(core_axis_name="c", subcore_axis_name="s")

    @functools.partial(
        pl.kernel,
        out_type=jax.ShapeDtypeStruct((NCORE, NP, H), f32),
        mesh=mesh,
        scratch_types=[
            pltpu.VMEM((SB, CHUNK), i32),
            pltpu.VMEM((SB, CHUNK), i32),
            pltpu.VMEM((CHUNK, H), f32),
            pltpu.VMEM((CHUNK, H), f32),
            pltpu.VMEM_SHARED((NP, H), f32),
            pltpu.SemaphoreType.DMA,
            pltpu.SemaphoreType.DMA,
        ],
    )
    def k(hs_hbm, row_hbm, col_hbm, out_hbm, row_v, col_v, buf_a, buf_b,
          acc, sem_a, sem_b):
        c = lax.axis_index("c")
        s = lax.axis_index("s")
        wid = c * NSUB + s
        _fill(buf_a, CHUNK, H, 0.0)
        zcps = [
            pltpu.async_copy(buf_a,
                             acc.at[pl.ds(s * RPS + j * CHUNK, CHUNK)], sem_a)
            for j in range(RPS // CHUNK)
        ]
        for cp in zcps:
            cp.wait()
        plsc.subcore_barrier()

        for t in range(CPW // SB):
            base = wid * CPW + t * SB
            pltpu.sync_copy(row_hbm.at[pl.ds(base, SB)], row_v)
            pltpu.sync_copy(col_hbm.at[pl.ds(base, SB)], col_v)
            pltpu.async_copy(hs_hbm.at[row_v.at[0]], buf_a, sem_a).wait()

            def body(i, carry):
                cp = pltpu.async_copy(hs_hbm.at[row_v.at[i + 1]], buf_b, sem_b)
                pltpu.sync_copy(buf_a, acc.at[col_v.at[i]], add=True)
                cp.wait()
                cp = pltpu.async_copy(hs_hbm.at[row_v.at[i + 2]], buf_a, sem_a)
                pltpu.sync_copy(buf_b, acc.at[col_v.at[i + 1]], add=True)
                cp.wait()
                return carry
            lax.fori_loop(0, SB // 2 - 1, lambda u, cr: body(2 * u, cr), 0)
            cp = pltpu.async_copy(hs_hbm.at[row_v.at[SB - 1]], buf_b, sem_b)
            pltpu.sync_copy(buf_a, acc.at[col_v.at[SB - 2]], add=True)
            cp.wait()
            pltpu.sync_copy(buf_b, acc.at[col_v.at[SB - 1]], add=True)
        plsc.subcore_barrier()
        ocps = [None, None]
        for j in range(RPS // CHUNK):
            off = s * RPS + j * CHUNK
            buf = buf_a if j % 2 == 0 else buf_b
            sem = sem_a if j % 2 == 0 else sem_b
            if ocps[j % 2] is not None:
                ocps[j % 2].wait()
            pltpu.sync_copy(acc.at[pl.ds(off, CHUNK)], buf)
            ocps[j % 2] = pltpu.async_copy(buf, out_hbm.at[c, pl.ds(off, CHUNK)],
                                           sem)
        for cp in ocps:
            if cp is not None:
                cp.wait()

    return k(hs, row2d, col2d)


def _dinv_of(deg_ref):
    d = deg_ref[0, :, 0:1] + deg_ref[1, :, 0:1]
    return lax.rsqrt(1.0 + d)


def _tc_z1(x, W1):
    def body(x_ref, w_ref, out_ref):
        out_ref[...] = jnp.dot(x_ref[...], w_ref[...],
                               preferred_element_type=f32,
                               precision=_HIGH)
    return pl.pallas_call(
        body,
        grid=(NBLK,),
        in_specs=[
            pl.BlockSpec((BLK, H), lambda i: (i, 0)),
            pl.BlockSpec((H, H), lambda i: (0, 0)),
        ],
        out_specs=pl.BlockSpec((BLK, H), lambda i: (i, 0)),
        out_shape=jax.ShapeDtypeStruct((NP, H), f32),
    )(x, W1)


def _tc_scale(z1, deg2):
    def body(z_ref, deg_ref, hs_ref, dv_ref):
        dinv = _dinv_of(deg_ref)
        hs_ref[...] = dinv * z_ref[...]
        dv_ref[...] = dinv
    return pl.pallas_call(
        body,
        grid=(NBLK,),
        in_specs=[
            pl.BlockSpec((BLK, H), lambda i: (i, 0)),
            pl.BlockSpec((2, BLK, H), lambda i: (0, i, 0)),
        ],
        out_specs=[
            pl.BlockSpec((BLK, H), lambda i: (i, 0)),
            pl.BlockSpec((BLK, 1), lambda i: (i, 0)),
        ],
        out_shape=[
            jax.ShapeDtypeStruct((NP, H), f32),
            jax.ShapeDtypeStruct((NP, 1), f32),
        ],
    )(z1, deg2)


def _tc_mid(agg2, hs, hprev, dinv, b, Wn, sub):
    def body(agg_ref, hs_ref, hp_ref, dv_ref, b_ref, w_ref, h_ref, hsn_ref):
        dinv = dv_ref[...]
        agg = agg_ref[0] + agg_ref[1]
        h = jnp.maximum(dinv * (agg + hs_ref[...]) + b_ref[...], 0.0)
        if sub:
            h = h - hp_ref[...]
        h_ref[...] = h
        hsn_ref[...] = dinv * jnp.dot(h, w_ref[...],
                                      preferred_element_type=f32,
                                      precision=_HIGH)
    return pl.pallas_call(
        body,
        grid=(NBLK,),
        in_specs=[
            pl.BlockSpec((2, BLK, H), lambda i: (0, i, 0)),
            pl.BlockSpec((BLK, H), lambda i: (i, 0)),
            pl.BlockSpec((BLK, H), lambda i: (i, 0)),
            pl.BlockSpec((BLK, 1), lambda i: (i, 0)),
            pl.BlockSpec((1, H), lambda i: (0, 0)),
            pl.BlockSpec((H, H), lambda i: (0, 0)),
        ],
        out_specs=[
            pl.BlockSpec((BLK, H), lambda i: (i, 0)),
            pl.BlockSpec((BLK, H), lambda i: (i, 0)),
        ],
        out_shape=[
            jax.ShapeDtypeStruct((NP, H), f32),
            jax.ShapeDtypeStruct((NP, H), f32),
        ],
    )(agg2, hs, hprev, dinv, b, Wn)


def _tc_final(agg2, hs, hprev, dinv, batch2d, b, Wm1, bm1, Wm2, bm2):
    def body(agg_ref, hs_ref, hp_ref, dv_ref, bt_ref, b_ref,
             wm1_ref, bm1_ref, wm2_ref, bm2_ref, out_ref, sums, cnts):
        i = pl.program_id(0)

        @pl.when(i == 0)
        def _init():
            sums[...] = jnp.zeros((B, H), f32)
            cnts[...] = jnp.zeros((B, 1), f32)

        dinv = dv_ref[...]
        agg = agg_ref[0] + agg_ref[1]
        h = jnp.maximum(dinv * (agg + hs_ref[...]) + b_ref[...], 0.0)
        h = h - hp_ref[...]
        bt = bt_ref[...].reshape(1, BLK)
        onehot = (lax.broadcasted_iota(i32, (B, BLK), 0) == bt).astype(f32)
        sums[...] += jnp.dot(onehot, h, preferred_element_type=f32,
                             precision=_HIGH)
        cnts[...] += jnp.sum(onehot, axis=1, keepdims=True)

        @pl.when(i == NBLK - 1)
        def _finish():
            g = sums[...] / jnp.maximum(cnts[...], 1.0)
            g = g / jnp.sqrt(jnp.sum(g * g, axis=1, keepdims=True))
            g = jnp.maximum(jnp.dot(g, wm1_ref[...],
                                    preferred_element_type=f32,
                                    precision=_HIGH) + bm1_ref[...], 0.0)
            g = g / jnp.sqrt(jnp.sum(g * g, axis=1, keepdims=True))
            g = jnp.dot(g, wm2_ref[...], preferred_element_type=f32,
                        precision=_HIGH) + bm2_ref[...]
            g = g / jnp.sqrt(jnp.sum(g * g, axis=1, keepdims=True))
            out_ref[...] = g

    return pl.pallas_call(
        body,
        grid=(NBLK,),
        in_specs=[
            pl.BlockSpec((2, BLK, H), lambda i: (0, i, 0)),
            pl.BlockSpec((BLK, H), lambda i: (i, 0)),
            pl.BlockSpec((BLK, H), lambda i: (i, 0)),
            pl.BlockSpec((BLK, 1), lambda i: (i, 0)),
            pl.BlockSpec((BLK, 1), lambda i: (i, 0)),
            pl.BlockSpec((1, H), lambda i: (0, 0)),
            pl.BlockSpec((H, NHID), lambda i: (0, 0)),
            pl.BlockSpec((1, NHID), lambda i: (0, 0)),
            pl.BlockSpec((NHID, H), lambda i: (0, 0)),
            pl.BlockSpec((1, H), lambda i: (0, 0)),
        ],
        out_specs=pl.BlockSpec((B, H), lambda i: (0, 0)),
        out_shape=jax.ShapeDtypeStruct((B, H), f32),
        scratch_shapes=[
            pltpu.VMEM((B, H), f32),
            pltpu.VMEM((B, 1), f32),
        ],
    )(agg2, hs, hprev, dinv, batch2d, b, Wm1, bm1, Wm2, bm2)


def kernel(x, edge_index, batch, W1, b1, W2, b2, W3, b3, W4, b4, W5, b5,
           Wm1, bm1, Wm2, bm2):
    row = edge_index[0].astype(i32)
    col = edge_index[1].astype(i32)
    fill = N + jnp.arange(EPAD - E, dtype=i32) % (NP - N)
    row2d = jnp.concatenate([row, fill]).reshape(NCHUNKS, CHUNK)
    col2d = jnp.concatenate([col, fill]).reshape(NCHUNKS, CHUNK)
    x_pad = jnp.pad(x, ((0, NP - N), (0, 0)))
    batch2d = jnp.pad(batch.astype(i32), (0, NP - N),
                      constant_values=B).reshape(NP, 1)
    b1r, b2r, b3r, b4r, b5r = (v.reshape(1, H) for v in (b1, b2, b3, b4, b5))
    bm1r = bm1.reshape(1, NHID)
    bm2r = bm2.reshape(1, H)

    z1 = _tc_z1(x_pad, W1)
    deg2 = _sc_deg(col2d)
    hs1, dinv = _tc_scale(z1, deg2)
    agg1 = _sc_agg(hs1, row2d, col2d)
    h1, hs2 = _tc_mid(agg1, hs1, x_pad, dinv, b1r, W2, sub=False)
    agg2 = _sc_agg(hs2, row2d, col2d)
    h2, hs3 = _tc_mid(agg2, hs2, h1, dinv, b2r, W3, sub=False)
    agg3 = _sc_agg(hs3, row2d, col2d)
    h3, hs4 = _tc_mid(agg3, hs3, h2, dinv, b3r, W4, sub=False)
    agg4 = _sc_agg(hs4, row2d, col2d)
    h4, hs5 = _tc_mid(agg4, hs4, h3, dinv, b4r, W5, sub=True)
    agg5 = _sc_agg(hs5, row2d, col2d)
    return _tc_final(agg5, hs5, h4, dinv, batch2d, b5r, Wm1, bm1r, Wm2, bm2r)


# speedup vs baseline: 20.0726x; 1.0094x over previous
import functools

import jax
import jax.numpy as jnp
from jax import lax
from jax.experimental import pallas as pl
from jax.experimental.pallas import tpu as pltpu
from jax.experimental.pallas import tpu_sc as plsc

N = 10000
NP = 10240
E = 320000
CHUNK = 128
NCHUNKS = 2560
EPAD = NCHUNKS * CHUNK
NSUB = 16
NCORE = 2
CPW = NCHUNKS // (NSUB * NCORE)
SB = 40
H = 128
NHID = 256
B = 256
RPS = NP // NSUB
DUMMY = N + 16
BLK = 1024
NBLK = NP // BLK

f32 = jnp.float32
i32 = jnp.int32
_HIGH = lax.Precision.HIGHEST


def _fill(ref, nrows, width, value):
    def body(i, carry):
        for j in range(width // 16):
            ref[i, pl.ds(j * 16, 16)] = jnp.full((16,), value, f32)
        return carry
    lax.fori_loop(0, nrows, body, 0)


def _sc_deg(col2d):
    mesh = plsc.VectorSubcoreMesh(core_axis_name="c", subcore_axis_name="s")

    @functools.partial(
        pl.kernel,
        out_type=jax.ShapeDtypeStruct((NCORE, NP, H), f32),
        mesh=mesh,
        scratch_types=[
            pltpu.VMEM((CPW, CHUNK), i32),
            pltpu.VMEM((CHUNK, H), f32),
            pltpu.VMEM((CHUNK, H), f32),
            pltpu.VMEM_SHARED((NP, H), f32),
        ],
    )
    def k(col_hbm, out_hbm, col_v, zb, ob, acc):
        c = lax.axis_index("c")
        s = lax.axis_index("s")
        wid = c * NSUB + s
        pltpu.sync_copy(col_hbm.at[pl.ds(wid * CPW, CPW)], col_v)
        _fill(zb, CHUNK, H, 0.0)
        _fill(ob, CHUNK, H, 1.0)
        for j in range(RPS // CHUNK):
            pltpu.sync_copy(zb, acc.at[pl.ds(s * RPS + j * CHUNK, CHUNK)])
        plsc.subcore_barrier()

        def body(i, carry):
            pltpu.sync_copy(ob, acc.at[col_v.at[i]], add=True)
            return carry
        lax.fori_loop(0, CPW, body, 0)
        plsc.subcore_barrier()
        for j in range(RPS // CHUNK):
            off = s * RPS + j * CHUNK
            pltpu.sync_copy(acc.at[pl.ds(off, CHUNK)], zb)
            pltpu.sync_copy(zb, out_hbm.at[c, pl.ds(off, CHUNK)])

    return k(col2d)


def _sc_agg(hs, row2d, col2d):
    mesh = plsc.VectorSubcoreMesh(core_axis_name="c", subcore_axis_name="s")

    @functools.partial(
        pl.kernel,
        out_type=jax.ShapeDtypeStruct((NCORE, NP, H), f32),
        mesh=mesh,
        scratch_types=[
            pltpu.VMEM((SB, CHUNK), i32),
            pltpu.VMEM((SB, CHUNK), i32),
            pltpu.VMEM((CHUNK, H), f32),
            pltpu.VMEM((CHUNK, H), f32),
            pltpu.VMEM_SHARED((NP, H), f32),
            pltpu.SemaphoreType.DMA,
            pltpu.SemaphoreType.DMA,
        ],
    )
    def k(hs_hbm, row_hbm, col_hbm, out_hbm, row_v, col_v, buf_a, buf_b,
          acc, sem_a, sem_b):
        c = lax.axis_index("c")
        s = lax.axis_index("s")
        wid = c * NSUB + s
        _fill(buf_a, CHUNK, H, 0.0)
        zcps = [
            pltpu.async_copy(buf_a,
                             acc.at[pl.ds(s * RPS + j * CHUNK, CHUNK)], sem_a)
            for j in range(RPS // CHUNK)
        ]
        for cp in zcps:
            cp.wait()
        plsc.subcore_barrier()

        for t in range(CPW // SB):
            base = wid * CPW + t * SB
            pltpu.sync_copy(row_hbm.at[pl.ds(base, SB)], row_v)
            pltpu.sync_copy(col_hbm.at[pl.ds(base, SB)], col_v)
            pltpu.async_copy(hs_hbm.at[row_v.at[0]], buf_a, sem_a).wait()

            def body(i, carry):
                cp = pltpu.async_copy(hs_hbm.at[row_v.at[i + 1]], buf_b, sem_b)
                pltpu.sync_copy(buf_a, acc.at[col_v.at[i]], add=True)
                cp.wait()
                cp = pltpu.async_copy(hs_hbm.at[row_v.at[i + 2]], buf_a, sem_a)
                pltpu.sync_copy(buf_b, acc.at[col_v.at[i + 1]], add=True)
                cp.wait()
                return carry
            lax.fori_loop(0, SB // 2 - 1, lambda u, cr: body(2 * u, cr), 0)
            cp = pltpu.async_copy(hs_hbm.at[row_v.at[SB - 1]], buf_b, sem_b)
            pltpu.sync_copy(buf_a, acc.at[col_v.at[SB - 2]], add=True)
            cp.wait()
            pltpu.sync_copy(buf_b, acc.at[col_v.at[SB - 1]], add=True)
        plsc.subcore_barrier()
        ocps = [None, None]
        for j in range(RPS // CHUNK):
            off = s * RPS + j * CHUNK
            buf = buf_a if j % 2 == 0 else buf_b
            sem = sem_a if j % 2 == 0 else sem_b
            if ocps[j % 2] is not None:
                ocps[j % 2].wait()
            pltpu.sync_copy(acc.at[pl.ds(off, CHUNK)], buf)
            ocps[j % 2] = pltpu.async_copy(buf, out_hbm.at[c, pl.ds(off, CHUNK)],
                                           sem)
        for cp in ocps:
            if cp is not None:
                cp.wait()

    return k(hs, row2d, col2d)


def _dinv_of(deg_ref):
    d = deg_ref[0, :, 0:1] + deg_ref[1, :, 0:1]
    return lax.rsqrt(1.0 + d)


def _tc_z1(x, W1):
    def body(x_ref, w_ref, out_ref):
        out_ref[...] = jnp.dot(x_ref[...], w_ref[...],
                               preferred_element_type=f32,
                               precision=_HIGH)
    return pl.pallas_call(
        body,
        grid=(NBLK,),
        in_specs=[
            pl.BlockSpec((BLK, H), lambda i: (i, 0)),
            pl.BlockSpec((H, H), lambda i: (0, 0)),
        ],
        out_specs=pl.BlockSpec((BLK, H), lambda i: (i, 0)),
        out_shape=jax.ShapeDtypeStruct((NP, H), f32),
    )(x, W1)


def _tc_scale(z1, deg2):
    def body(z_ref, deg_ref, hs_ref, dv_ref):
        dinv = _dinv_of(deg_ref)
        hs_ref[...] = dinv * z_ref[...]
        dv_ref[...] = dinv
    return pl.pallas_call(
        body,
        grid=(NBLK,),
        in_specs=[
            pl.BlockSpec((BLK, H), lambda i: (i, 0)),
            pl.BlockSpec((2, BLK, H), lambda i: (0, i, 0)),
        ],
        out_specs=[
            pl.BlockSpec((BLK, H), lambda i: (i, 0)),
            pl.BlockSpec((BLK, 1), lambda i: (i, 0)),
        ],
        out_shape=[
            jax.ShapeDtypeStruct((NP, H), f32),
            jax.ShapeDtypeStruct((NP, 1), f32),
        ],
    )(z1, deg2)


def _tc_mid(agg2, hs, hprev, dinv, b, Wn, sub, need_h):
    def body(*refs):
        if sub:
            agg_ref, hs_ref, hp_ref, dv_ref, b_ref, w_ref = refs[:6]
        else:
            agg_ref, hs_ref, dv_ref, b_ref, w_ref = refs[:5]
        out_refs = refs[6 if sub else 5:]
        dinv = dv_ref[...]
        agg = agg_ref[0] + agg_ref[1]
        h = jnp.maximum(dinv * (agg + hs_ref[...]) + b_ref[...], 0.0)
        if sub:
            h = h - hp_ref[...]
        if need_h:
            out_refs[0][...] = h
        out_refs[-1][...] = dinv * jnp.dot(h, w_ref[...],
                                           preferred_element_type=f32,
                                           precision=_HIGH)
    blk = pl.BlockSpec((BLK, H), lambda i: (i, 0))
    in_specs = [pl.BlockSpec((2, BLK, H), lambda i: (0, i, 0)), blk]
    args = [agg2, hs]
    if sub:
        in_specs.append(blk)
        args.append(hprev)
    in_specs += [
        pl.BlockSpec((BLK, 1), lambda i: (i, 0)),
        pl.BlockSpec((1, H), lambda i: (0, 0)),
        pl.BlockSpec((H, H), lambda i: (0, 0)),
    ]
    args += [dinv, b, Wn]
    n_out = 2 if need_h else 1
    outs = pl.pallas_call(
        body,
        grid=(NBLK,),
        in_specs=in_specs,
        out_specs=[blk] * n_out,
        out_shape=[jax.ShapeDtypeStruct((NP, H), f32)] * n_out,
    )(*args)
    return (outs[0], outs[1]) if need_h else (None, outs[0])


def _tc_final(agg2, hs, hprev, dinv, batch2d, b, Wm1, bm1, Wm2, bm2):
    def body(agg_ref, hs_ref, hp_ref, dv_ref, bt_ref, b_ref,
             wm1_ref, bm1_ref, wm2_ref, bm2_ref, out_ref, sums, cnts):
        i = pl.program_id(0)

        @pl.when(i == 0)
        def _init():
            sums[...] = jnp.zeros((B, H), f32)
            cnts[...] = jnp.zeros((B, 1), f32)

        dinv = dv_ref[...]
        agg = agg_ref[0] + agg_ref[1]
        h = jnp.maximum(dinv * (agg + hs_ref[...]) + b_ref[...], 0.0)
        h = h - hp_ref[...]
        bt = bt_ref[...].reshape(1, BLK)
        onehot = (lax.broadcasted_iota(i32, (B, BLK), 0) == bt).astype(f32)
        sums[...] += jnp.dot(onehot, h, preferred_element_type=f32,
                             precision=_HIGH)
        cnts[...] += jnp.sum(onehot, axis=1, keepdims=True)

        @pl.when(i == NBLK - 1)
        def _finish():
            g = sums[...] / jnp.maximum(cnts[...], 1.0)
            g = g / jnp.sqrt(jnp.sum(g * g, axis=1, keepdims=True))
            g = jnp.maximum(jnp.dot(g, wm1_ref[...],
                                    preferred_element_type=f32,
                                    precision=_HIGH) + bm1_ref[...], 0.0)
            g = g / jnp.sqrt(jnp.sum(g * g, axis=1, keepdims=True))
            g = jnp.dot(g, wm2_ref[...], preferred_element_type=f32,
                        precision=_HIGH) + bm2_ref[...]
            g = g / jnp.sqrt(jnp.sum(g * g, axis=1, keepdims=True))
            out_ref[...] = g

    return pl.pallas_call(
        body,
        grid=(NBLK,),
        in_specs=[
            pl.BlockSpec((2, BLK, H), lambda i: (0, i, 0)),
            pl.BlockSpec((BLK, H), lambda i: (i, 0)),
            pl.BlockSpec((BLK, H), lambda i: (i, 0)),
            pl.BlockSpec((BLK, 1), lambda i: (i, 0)),
            pl.BlockSpec((BLK, 1), lambda i: (i, 0)),
            pl.BlockSpec((1, H), lambda i: (0, 0)),
            pl.BlockSpec((H, NHID), lambda i: (0, 0)),
            pl.BlockSpec((1, NHID), lambda i: (0, 0)),
            pl.BlockSpec((NHID, H), lambda i: (0, 0)),
            pl.BlockSpec((1, H), lambda i: (0, 0)),
        ],
        out_specs=pl.BlockSpec((B, H), lambda i: (0, 0)),
        out_shape=jax.ShapeDtypeStruct((B, H), f32),
        scratch_shapes=[
            pltpu.VMEM((B, H), f32),
            pltpu.VMEM((B, 1), f32),
        ],
    )(agg2, hs, hprev, dinv, batch2d, b, Wm1, bm1, Wm2, bm2)


def kernel(x, edge_index, batch, W1, b1, W2, b2, W3, b3, W4, b4, W5, b5,
           Wm1, bm1, Wm2, bm2):
    row = edge_index[0].astype(i32)
    col = edge_index[1].astype(i32)
    fill = N + jnp.arange(EPAD - E, dtype=i32) % (NP - N)
    row2d = jnp.concatenate([row, fill]).reshape(NCHUNKS, CHUNK)
    col2d = jnp.concatenate([col, fill]).reshape(NCHUNKS, CHUNK)
    x_pad = jnp.pad(x, ((0, NP - N), (0, 0)))
    batch2d = jnp.pad(batch.astype(i32), (0, NP - N),
                      constant_values=B).reshape(NP, 1)
    b1r, b2r, b3r, b4r, b5r = (v.reshape(1, H) for v in (b1, b2, b3, b4, b5))
    bm1r = bm1.reshape(1, NHID)
    bm2r = bm2.reshape(1, H)

    z1 = _tc_z1(x_pad, W1)
    deg2 = _sc_deg(col2d)
    hs1, dinv = _tc_scale(z1, deg2)
    agg1 = _sc_agg(hs1, row2d, col2d)
    _, hs2 = _tc_mid(agg1, hs1, None, dinv, b1r, W2, sub=False, need_h=False)
    agg2 = _sc_agg(hs2, row2d, col2d)
    _, hs3 = _tc_mid(agg2, hs2, None, dinv, b2r, W3, sub=False, need_h=False)
    agg3 = _sc_agg(hs3, row2d, col2d)
    h3, hs4 = _tc_mid(agg3, hs3, None, dinv, b3r, W4, sub=False, need_h=True)
    agg4 = _sc_agg(hs4, row2d, col2d)
    h4, hs5 = _tc_mid(agg4, hs4, h3, dinv, b4r, W5, sub=True, need_h=True)
    agg5 = _sc_agg(hs5, row2d, col2d)
    return _tc_final(agg5, hs5, h4, dinv, batch2d, b5r, Wm1, bm1r, Wm2, bm2r)
